# baseline (device time: 532520 ns/iter reference)
import jax
import jax.numpy as jnp
from jax import lax
from jax.experimental import pallas as pl
from jax.experimental.pallas import tpu as pltpu

T = 2048
D = 4096
V_HALF = 8192
V = 2 * V_HALF
CB = 512
NCB = V_HALF // CB
assert NCB <= 16

ML = 16
SL = 17


def _fused(x_bf16, W):
    def body(
        x_ref,
        W_ref,
        out_ref,
        sbuf,
        rbuf,
        wstage,
        wbuf,
        ebuf,
        rconv,
        obuf,
        stats_mine,
        stats_theirs,
        scales,
        mrun, srun,
        wload_sem, estage_sem, rload_sems, ostore_sems,
        stat_send_sem, stat_recv_sem,
        send_sems, recv_sems,
    ):
        my_x = lax.axis_index("x")
        my_y = lax.axis_index("y")
        nbr = (1 - my_x, my_y)

        barrier = pltpu.get_barrier_semaphore()
        pl.semaphore_signal(
            barrier, inc=1, device_id=nbr, device_id_type=pl.DeviceIdType.MESH
        )
        pl.semaphore_wait(barrier, 1)

        def wload(cb):
            return pltpu.make_async_copy(
                W_ref.at[:, pl.ds(cb * CB, CB)], wstage, wload_sem
            )

        def chunk_rdma(cb):
            return pltpu.make_async_remote_copy(
                src_ref=sbuf.at[cb],
                dst_ref=rbuf.at[cb],
                send_sem=send_sems.at[cb],
                recv_sem=recv_sems.at[cb],
                device_id=nbr,
                device_id_type=pl.DeviceIdType.MESH,
            )

        mrun[...] = jnp.full((T, 1), -jnp.inf, jnp.float32)
        srun[...] = jnp.zeros((T, 1), jnp.float32)
        wload(0).start()

        def phase1(cb, carry):
            wload(cb).wait()
            wbuf[...] = wstage[...].astype(jnp.bfloat16)

            @pl.when(cb + 1 < NCB)
            def _():
                wload(cb + 1).start()

            logits = jnp.dot(
                x_ref[...], wbuf[...], preferred_element_type=jnp.float32
            )
            m_cb = jnp.max(logits, axis=1, keepdims=True)
            e = jnp.exp(logits - m_cb)
            s_cb = jnp.sum(e, axis=1, keepdims=True)
            lane = lax.broadcasted_iota(jnp.int32, (T, 32), 1)
            stats_mine[...] = jnp.where(lane == cb, m_cb, stats_mine[...])
            m_new = jnp.maximum(mrun[...], m_cb)
            srun[...] = srun[...] * jnp.exp(mrun[...] - m_new) + s_cb * jnp.exp(
                m_cb - m_new
            )
            mrun[...] = m_new
            ebuf[...] = e.astype(jnp.bfloat16)
            stage = pltpu.make_async_copy(ebuf, sbuf.at[cb], estage_sem)
            stage.start()
            stage.wait()
            chunk_rdma(cb).start()
            return carry

        lax.fori_loop(0, NCB, phase1, 0)

        stats_mine[:, ML : ML + 1] = mrun[...]
        stats_mine[:, SL : SL + 1] = srun[...]
        st = pltpu.make_async_remote_copy(
            src_ref=stats_mine,
            dst_ref=stats_theirs,
            send_sem=stat_send_sem,
            recv_sem=stat_recv_sem,
            device_id=nbr,
            device_id_type=pl.DeviceIdType.MESH,
        )
        st.start()
        st.wait()

        m_loc = stats_mine[:, ML : ML + 1]
        s_loc = stats_mine[:, SL : SL + 1]
        m_rem = stats_theirs[:, ML : ML + 1]
        s_rem = stats_theirs[:, SL : SL + 1]
        mm = jnp.maximum(m_loc, m_rem)
        ss = s_loc * jnp.exp(m_loc - mm) + s_rem * jnp.exp(m_rem - mm)
        scales[:, 0:NCB] = jnp.exp(stats_mine[:, 0:NCB] - mm) / ss
        scales[:, 16 : 16 + NCB] = jnp.exp(stats_theirs[:, 0:NCB] - mm) / ss

        def store_dma(slot, cb, col0):
            return pltpu.make_async_copy(
                obuf.at[slot],
                out_ref.at[:, pl.ds(col0 + cb * CB, CB)],
                ostore_sems.at[slot],
            )

        def emit_loop(src_hbm, lane0, col0, wait_recv_first):
            def eload(cb):
                return pltpu.make_async_copy(
                    src_hbm.at[cb],
                    rconv.at[lax.rem(cb, 2)],
                    rload_sems.at[lax.rem(cb, 2)],
                )

            if wait_recv_first:
                chunk_rdma(0).wait_recv()
            eload(0).start()

            def emit(cb, carry):
                slot = lax.rem(cb, 2)

                @pl.when(cb >= 2)
                def _():
                    store_dma(slot, cb - 2, col0).wait()

                eload(cb).wait()

                @pl.when(cb + 1 < NCB)
                def _():
                    if wait_recv_first:
                        chunk_rdma(cb + 1).wait_recv()
                    eload(cb + 1).start()

                lane = lax.broadcasted_iota(jnp.int32, (T, 32), 1)
                svec = jnp.sum(
                    jnp.where(lane == lane0 + cb, scales[...], 0.0),
                    axis=1,
                    keepdims=True,
                )
                obuf[slot] = rconv[slot] * svec.astype(jnp.bfloat16)
                store_dma(slot, cb, col0).start()
                return carry

            lax.fori_loop(0, NCB, emit, 0)
            for last in (NCB - 2, NCB - 1):
                store_dma(last % 2, last, col0).wait()

        emit_loop(sbuf, 0, my_x * V_HALF, False)
        emit_loop(rbuf, 16, (1 - my_x) * V_HALF, True)

        def waitsend(cb, carry):
            chunk_rdma(cb).wait_send()
            return carry

        lax.fori_loop(0, NCB, waitsend, 0)

    out = pl.pallas_call(
        body,
        out_shape=[
            jax.ShapeDtypeStruct((T, V), jnp.bfloat16),
            jax.ShapeDtypeStruct((NCB, T, CB), jnp.bfloat16),
            jax.ShapeDtypeStruct((NCB, T, CB), jnp.bfloat16),
        ],
        in_specs=[
            pl.BlockSpec(memory_space=pltpu.MemorySpace.VMEM),
            pl.BlockSpec(memory_space=pltpu.MemorySpace.HBM),
        ],
        out_specs=[
            pl.BlockSpec(memory_space=pltpu.MemorySpace.HBM),
            pl.BlockSpec(memory_space=pltpu.MemorySpace.HBM),
            pl.BlockSpec(memory_space=pltpu.MemorySpace.HBM),
        ],
        scratch_shapes=[
            pltpu.VMEM((D, CB), jnp.float32),
            pltpu.VMEM((D, CB), jnp.bfloat16),
            pltpu.VMEM((T, CB), jnp.bfloat16),
            pltpu.VMEM((2, T, CB), jnp.bfloat16),
            pltpu.VMEM((2, T, CB), jnp.bfloat16),
            pltpu.VMEM((T, 32), jnp.float32),
            pltpu.VMEM((T, 32), jnp.float32),
            pltpu.VMEM((T, 32), jnp.float32),
            pltpu.VMEM((T, 1), jnp.float32),
            pltpu.VMEM((T, 1), jnp.float32),
            pltpu.SemaphoreType.DMA,
            pltpu.SemaphoreType.DMA,
            pltpu.SemaphoreType.DMA((2,)),
            pltpu.SemaphoreType.DMA((2,)),
            pltpu.SemaphoreType.DMA,
            pltpu.SemaphoreType.DMA,
            pltpu.SemaphoreType.DMA((NCB,)),
            pltpu.SemaphoreType.DMA((NCB,)),
        ],
        compiler_params=pltpu.CompilerParams(
            collective_id=0, vmem_limit_bytes=56 * 1024 * 1024
        ),
    )(x_bf16, W)
    return out[0]


def kernel(x, W):
    return _fused(x.astype(jnp.bfloat16), W)
